# Initial kernel scaffold; baseline (speedup 1.0000x reference)
import jax
import jax.numpy as jnp
from jax.experimental import pallas as pl


def kernel(adj, final_features, segment, idx, W0, b0, W1, b1, gamma1, beta1, gamma2, beta2, fc1_W, fc1_b):
    raise NotImplementedError("write your pallas kernel here")


# speedup vs baseline: 1.3592x; 1.3592x over previous
import functools

import jax
import jax.numpy as jnp
from jax.experimental import pallas as pl

N = 10000
D = 128
G = 512
TR = 400


def _xw_kernel(x_ref, w_ref, out_ref):
    out_ref[...] = jnp.dot(x_ref[...], w_ref[...],
                           preferred_element_type=jnp.float32)


def _mp_a_kernel(adj_ref, z_ref, b_ref, w_ref, out_ref):
    acc = jnp.dot(adj_ref[...], z_ref[...],
                  preferred_element_type=jnp.float32)
    h = jnp.maximum(acc + b_ref[...], 0.0)
    out_ref[...] = jnp.dot(h, w_ref[...], preferred_element_type=jnp.float32)


def _mp_b_kernel(adj_ref, z_ref, b_ref, out_ref):
    acc = jnp.dot(adj_ref[...], z_ref[...],
                  preferred_element_type=jnp.float32)
    out_ref[...] = jnp.maximum(acc + b_ref[...], 0.0)


def _tail_kernel(x_ref, idx_ref, g1_ref, be1_ref, g2_ref, be2_ref,
                 fw_ref, fb_ref, out_ref):
    x = x_ref[...]
    mean1 = jnp.mean(x, axis=0, keepdims=True)
    var1 = jnp.mean((x - mean1) ** 2, axis=0, keepdims=True)
    xn = (x - mean1) / jnp.sqrt(var1 + 1e-5) * g1_ref[...] + be1_ref[...]
    ids = idx_ref[...]
    gi = jax.lax.broadcasted_iota(jnp.int32, (G, N), 0)
    onehot = (gi == ids).astype(jnp.float32)
    pooled = jnp.dot(onehot, xn, preferred_element_type=jnp.float32)
    mean2 = jnp.mean(pooled, axis=0, keepdims=True)
    var2 = jnp.mean((pooled - mean2) ** 2, axis=0, keepdims=True)
    y = (pooled - mean2) / jnp.sqrt(var2 + 1e-5) * g2_ref[...] + be2_ref[...]
    out = jnp.dot(y, fw_ref[...], preferred_element_type=jnp.float32)
    out_ref[...] = jnp.maximum(out + fb_ref[...], 0.0)


def _full(shape):
    return pl.BlockSpec(shape, lambda *_: tuple(0 for _ in shape))


@functools.partial(jax.jit, static_argnames=("interpret",))
def _run(adj, x, idx, W0, b0, W1, b1, gamma1, beta1, gamma2, beta2,
         fc1_W, fc1_b, interpret=False):
    f32 = jnp.float32
    z0 = pl.pallas_call(
        _xw_kernel,
        out_shape=jax.ShapeDtypeStruct((N, D), f32),
        interpret=interpret,
    )(x, W0)

    row = pl.BlockSpec((TR, N), lambda i: (i, 0))
    outrow = pl.BlockSpec((TR, D), lambda i: (i, 0))
    z1 = pl.pallas_call(
        _mp_a_kernel,
        grid=(N // TR,),
        in_specs=[row, _full((N, D)), _full((1, D)), _full((D, D))],
        out_specs=outrow,
        out_shape=jax.ShapeDtypeStruct((N, D), f32),
        interpret=interpret,
    )(adj, z0, b0.reshape(1, D), W1)

    h2 = pl.pallas_call(
        _mp_b_kernel,
        grid=(N // TR,),
        in_specs=[row, _full((N, D)), _full((1, D))],
        out_specs=outrow,
        out_shape=jax.ShapeDtypeStruct((N, D), f32),
        interpret=interpret,
    )(adj, z1, b1.reshape(1, D))

    out = pl.pallas_call(
        _tail_kernel,
        out_shape=jax.ShapeDtypeStruct((G, D), f32),
        interpret=interpret,
    )(h2, idx.reshape(1, N).astype(jnp.int32),
      gamma1.reshape(1, D), beta1.reshape(1, D),
      gamma2.reshape(1, D), beta2.reshape(1, D), fc1_W,
      fc1_b.reshape(1, D))
    return out


def kernel(adj, final_features, segment, idx, W0, b0, W1, b1,
           gamma1, beta1, gamma2, beta2, fc1_W, fc1_b):
    return _run(adj, final_features, idx, W0, b0, W1, b1,
                gamma1, beta1, gamma2, beta2, fc1_W, fc1_b)
